# baseline (speedup 1.0000x reference)
import jax
import jax.numpy as jnp
from jax.experimental import pallas as pl
from jax.experimental.pallas import tpu as pltpu

_SCALING = 16.0 / 8.0


def _fused_kernel(x_ref, w_ref, sm_ref, b_ref, bb_ref, o_ref, wb_ref,
                  xb_ref):
    i = pl.program_id(0)
    j = pl.program_id(1)

    @pl.when(jnp.logical_and(i == 0, j == 0))
    def _cast_w():
        wb_ref[...] = w_ref[...].astype(jnp.bfloat16)

    @pl.when(j == 0)
    def _pack_x():
        xb_ref[...] = x_ref[...].astype(jnp.bfloat16)

    @pl.when(j == 1)
    def _compute():
        xt = xb_ref[...]
        small = jax.lax.dot_general(
            xt, sm_ref[...], (((1,), (1,)), ((), ())),
            preferred_element_type=jnp.float32)
        xa = small[:, 0:8]
        l0 = small[:, 8:9]
        l1 = small[:, 9:10]
        w = jnp.where(l0 >= l1, jax.nn.sigmoid(l0 - l1), 0.0) * _SCALING
        xa16 = (xa * w).astype(jnp.bfloat16)
        base = jax.lax.dot_general(
            xt, wb_ref[...], (((1,), (1,)), ((), ())),
            preferred_element_type=jnp.float32)
        lora = jax.lax.dot_general(
            xa16, bb_ref[...], (((1,), (1,)), ((), ())),
            preferred_element_type=jnp.float32)
        o_ref[...] = base + b_ref[...] + lora


def kernel(x, base_W, base_b, gate_W, lora_A_W, lora_B_W):
    n_tokens, in_f = x.shape
    out_f = base_W.shape[0]
    tm = 512
    grid = (n_tokens // tm, 2)
    small_W = jnp.concatenate(
        [lora_A_W, gate_W, jnp.zeros((6, in_f), jnp.float32)],
        axis=0).astype(jnp.bfloat16)
    bias2d = base_b.reshape(1, out_f)
    return pl.pallas_call(
        _fused_kernel,
        grid=grid,
        in_specs=[
            pl.BlockSpec((tm, in_f), lambda i, j: (i, 0)),
            pl.BlockSpec((out_f, in_f), lambda i, j: (0, 0)),
            pl.BlockSpec((16, in_f), lambda i, j: (0, 0)),
            pl.BlockSpec((1, out_f), lambda i, j: (0, 0)),
            pl.BlockSpec((out_f, 8), lambda i, j: (0, 0)),
        ],
        out_specs=pl.BlockSpec((tm, out_f), lambda i, j: (i, 0)),
        out_shape=jax.ShapeDtypeStruct((n_tokens, out_f), jnp.float32),
        scratch_shapes=[pltpu.VMEM((out_f, in_f), jnp.bfloat16),
                        pltpu.VMEM((tm, in_f), jnp.bfloat16)],
        compiler_params=pltpu.CompilerParams(
            dimension_semantics=(pltpu.ARBITRARY, pltpu.ARBITRARY)),
    )(x, base_W, small_W, bias2d, lora_B_W.astype(jnp.bfloat16))


# speedup vs baseline: 1.1421x; 1.1421x over previous
import jax
import jax.numpy as jnp
from jax.experimental import pallas as pl
from jax.experimental.pallas import tpu as pltpu

_SCALING = 16.0 / 8.0


def _make_kernel(tm):
    def _phase(x_ref, sm_ref, b_ref, bb_ref, o_ref, wb_ref, wr_ref, rd_ref):
        wr_ref[...] = x_ref[...].astype(jnp.bfloat16)
        xt = rd_ref[...]
        small = jax.lax.dot_general(
            xt, sm_ref[...], (((1,), (1,)), ((), ())),
            preferred_element_type=jnp.float32)
        xa = small[:, 0:8]
        l0 = small[:, 8:9]
        l1 = small[:, 9:10]
        w = jnp.where(l0 >= l1, jax.nn.sigmoid(l0 - l1), 0.0) * _SCALING
        xa16 = (xa * w).astype(jnp.bfloat16)
        base = jax.lax.dot_general(
            xt, wb_ref[...], (((1,), (1,)), ((), ())),
            preferred_element_type=jnp.float32)
        lora = jax.lax.dot_general(
            xa16, bb_ref[...], (((1,), (1,)), ((), ())),
            preferred_element_type=jnp.float32)
        o_ref[...] = base + b_ref[...] + lora

    def _fused_kernel(x_ref, w_ref, sm_ref, b_ref, bb_ref, o_ref, wb_ref,
                      xb0_ref, xb1_ref):
        i = pl.program_id(0)

        @pl.when(i == 0)
        def _cast_w():
            wb_ref[...] = w_ref[...].astype(jnp.bfloat16)

        par = jax.lax.rem(i, 2)

        @pl.when(par == 0)
        def _even():
            _phase(x_ref, sm_ref, b_ref, bb_ref, o_ref, wb_ref,
                   xb0_ref, xb1_ref)

        @pl.when(par == 1)
        def _odd():
            _phase(x_ref, sm_ref, b_ref, bb_ref, o_ref, wb_ref,
                   xb1_ref, xb0_ref)

    return _fused_kernel


def kernel(x, base_W, base_b, gate_W, lora_A_W, lora_B_W):
    n_tokens, in_f = x.shape
    out_f = base_W.shape[0]
    tm = 512
    n_tiles = n_tokens // tm
    grid = (n_tiles + 1,)
    small_W = jnp.concatenate(
        [lora_A_W, gate_W, jnp.zeros((6, in_f), jnp.float32)],
        axis=0).astype(jnp.bfloat16)
    bias2d = base_b.reshape(1, out_f)
    last = n_tiles - 1
    return pl.pallas_call(
        _make_kernel(tm),
        grid=grid,
        in_specs=[
            pl.BlockSpec((tm, in_f), lambda i: (jnp.minimum(i, last), 0)),
            pl.BlockSpec((out_f, in_f), lambda i: (0, 0)),
            pl.BlockSpec((16, in_f), lambda i: (0, 0)),
            pl.BlockSpec((1, out_f), lambda i: (0, 0)),
            pl.BlockSpec((out_f, 8), lambda i: (0, 0)),
        ],
        out_specs=pl.BlockSpec(
            (tm, out_f), lambda i: (jnp.maximum(i - 1, 0), 0)),
        out_shape=jax.ShapeDtypeStruct((n_tokens, out_f), jnp.float32),
        scratch_shapes=[pltpu.VMEM((out_f, in_f), jnp.bfloat16),
                        pltpu.VMEM((tm, in_f), jnp.bfloat16),
                        pltpu.VMEM((tm, in_f), jnp.bfloat16)],
        compiler_params=pltpu.CompilerParams(
            dimension_semantics=(pltpu.ARBITRARY,)),
    )(x, base_W, small_W, bias2d, lora_B_W.astype(jnp.bfloat16))


# speedup vs baseline: 1.3974x; 1.2236x over previous
import jax
import jax.numpy as jnp
from jax.experimental import pallas as pl
from jax.experimental.pallas import tpu as pltpu

_SCALING = 16.0 / 8.0


def _fused_kernel(x_ref, w_ref, b_ref, g_ref, a_ref, bb_ref, o_ref):
    xt = x_ref[...]
    base = jax.lax.dot_general(
        xt, w_ref[...], (((1,), (1,)), ((), ())),
        preferred_element_type=jnp.float32)
    logits = jax.lax.dot_general(
        xt, g_ref[...], (((1,), (1,)), ((), ())),
        preferred_element_type=jnp.float32)
    l0 = logits[:, 0:1]
    l1 = logits[:, 1:2]
    w = jnp.where(l0 >= l1, jax.nn.sigmoid(l0 - l1), 0.0) * _SCALING
    xa = jax.lax.dot_general(
        xt, a_ref[...], (((1,), (1,)), ((), ())),
        preferred_element_type=jnp.float32)
    xa = xa * w
    lora = jax.lax.dot_general(
        xa, bb_ref[...], (((1,), (1,)), ((), ())),
        preferred_element_type=jnp.float32)
    o_ref[...] = base + b_ref[...] + lora


def kernel(x, base_W, base_b, gate_W, lora_A_W, lora_B_W):
    n_tokens, in_f = x.shape
    out_f = base_W.shape[0]
    tm = 1024
    grid = (n_tokens // tm,)
    bias2d = base_b.reshape(1, out_f)
    return pl.pallas_call(
        _fused_kernel,
        grid=grid,
        in_specs=[
            pl.BlockSpec((tm, in_f), lambda i: (i, 0)),
            pl.BlockSpec((out_f, in_f), lambda i: (0, 0)),
            pl.BlockSpec((1, out_f), lambda i: (0, 0)),
            pl.BlockSpec(gate_W.shape, lambda i: (0, 0)),
            pl.BlockSpec(lora_A_W.shape, lambda i: (0, 0)),
            pl.BlockSpec(lora_B_W.shape, lambda i: (0, 0)),
        ],
        out_specs=pl.BlockSpec((tm, out_f), lambda i: (i, 0)),
        out_shape=jax.ShapeDtypeStruct((n_tokens, out_f), jnp.float32),
        compiler_params=pltpu.CompilerParams(
            dimension_semantics=(pltpu.PARALLEL,)),
    )(x, base_W, bias2d, gate_W, lora_A_W, lora_B_W)
